# baseline (speedup 1.0000x reference)
import functools

import jax
import jax.numpy as jnp
from jax import lax
from jax.experimental import pallas as pl
from jax.experimental.pallas import tpu as pltpu
from jax.experimental.pallas import tpu_sc as plsc

_NUM_CORES = 2
_NUM_SUBCORES = 16
_NUM_WORKERS = _NUM_CORES * _NUM_SUBCORES
_CHUNK = 32768
_NBUF = 3


def _sc_copy(k_cache1, v_cache1, *, total):
    elems_per = total // _NUM_WORKERS
    nchunk = elems_per // _CHUNK
    mesh = plsc.VectorSubcoreMesh(
        core_axis_name="c", subcore_axis_name="s",
        num_cores=_NUM_CORES, num_subcores=_NUM_SUBCORES)

    @functools.partial(
        pl.kernel,
        out_type=(
            jax.ShapeDtypeStruct((total,), jnp.float32),
            jax.ShapeDtypeStruct((total,), jnp.float32),
        ),
        mesh=mesh,
        scratch_types=[
            pltpu.VMEM((_CHUNK,), jnp.float32),
            pltpu.VMEM((_CHUNK,), jnp.float32),
            pltpu.VMEM((_CHUNK,), jnp.float32),
            pltpu.SemaphoreType.DMA,
            pltpu.SemaphoreType.DMA,
        ],
    )
    def body(kc_hbm, vc_hbm, kout_hbm, vout_hbm, buf0, buf1, buf2,
             sem_ld, sem_st):
        bufs = (buf0, buf1, buf2)
        wid = lax.axis_index("s") * _NUM_CORES + lax.axis_index("c")
        e0 = wid * elems_per

        work = []
        for c in range(nchunk):
            work.append((kc_hbm, kout_hbm, c))
            work.append((vc_hbm, vout_hbm, c))
        n = len(work)

        loads = [None] * n
        stores = [None] * n

        def start_load(i):
            src, _, c = work[i]
            cp = pltpu.make_async_copy(
                src.at[pl.ds(e0 + c * _CHUNK, _CHUNK)],
                bufs[i % _NBUF], sem_ld)
            cp.start()
            loads[i] = cp

        def start_store(i):
            _, dst, c = work[i]
            cp = pltpu.make_async_copy(
                bufs[i % _NBUF],
                dst.at[pl.ds(e0 + c * _CHUNK, _CHUNK)], sem_st)
            cp.start()
            stores[i] = cp

        for i in range(min(_NBUF, n)):
            start_load(i)
        for i in range(n):
            loads[i].wait()
            start_store(i)
            nxt = i + _NBUF
            if nxt < n:
                stores[i].wait()
                start_load(nxt)
        for i in range(max(0, n - _NBUF), n):
            if stores[i] is not None:
                stores[i].wait()

    return body(k_cache1, v_cache1)


def kernel(input_pos, k_val, v_val, k_cache, v_cache):
    B, H, Q, D = k_val.shape
    S = k_cache.shape[2]
    total = B * H * S * D
    k_out1, v_out1 = _sc_copy(
        k_cache.reshape(-1), v_cache.reshape(-1), total=total)
    return (k_out1.reshape(B, H, S, D), v_out1.reshape(B, H, S, D))


# speedup vs baseline: 1.0838x; 1.0838x over previous
import functools

import jax
import jax.numpy as jnp
from jax import lax
from jax.experimental import pallas as pl
from jax.experimental.pallas import tpu as pltpu
from jax.experimental.pallas import tpu_sc as plsc

_NUM_CORES = 2
_NUM_SUBCORES = 16
_NUM_WORKERS = _NUM_CORES * _NUM_SUBCORES
_CHUNK_ROWS = 256
_NBUF = 3
_TC_SLABS = 2


def _sc_k_path(pos, k_val2, k_cache2, *, n_slabs, S, Q, D):
    slabs_per = n_slabs // _NUM_WORKERS
    rows_per = slabs_per * S
    nchunk = rows_per // _CHUNK_ROWS
    nval = slabs_per * Q
    mesh = plsc.VectorSubcoreMesh(
        core_axis_name="c", subcore_axis_name="s",
        num_cores=_NUM_CORES, num_subcores=_NUM_SUBCORES)

    @functools.partial(
        pl.kernel,
        out_type=jax.ShapeDtypeStruct((n_slabs * S, D), jnp.float32),
        mesh=mesh,
        scratch_types=[
            pltpu.VMEM((_CHUNK_ROWS, D), jnp.float32),
            pltpu.VMEM((_CHUNK_ROWS, D), jnp.float32),
            pltpu.VMEM((_CHUNK_ROWS, D), jnp.float32),
            pltpu.VMEM((Q,), jnp.int32),
            pltpu.VMEM((nval, D), jnp.float32),
            pltpu.SemaphoreType.DMA,
            pltpu.SemaphoreType.DMA,
            pltpu.SemaphoreType.DMA,
        ],
    )
    def body(pos_hbm, kval_hbm, kc_hbm, kout_hbm,
             buf0, buf1, buf2, pos_v, kv_v, sem_ld, sem_st, sem_sc):
        bufs = (buf0, buf1, buf2)
        wid = lax.axis_index("s") * _NUM_CORES + lax.axis_index("c")
        base = wid * slabs_per
        r0 = base * S

        lk = pltpu.make_async_copy(
            kval_hbm.at[pl.ds(base * Q, nval)], kv_v, sem_sc)
        lk.start()
        pltpu.sync_copy(pos_hbm, pos_v)
        lk.wait()

        loads = [None] * nchunk
        stores = [None] * nchunk

        def start_load(i):
            cp = pltpu.make_async_copy(
                kc_hbm.at[pl.ds(r0 + i * _CHUNK_ROWS, _CHUNK_ROWS)],
                bufs[i % _NBUF], sem_ld)
            cp.start()
            loads[i] = cp

        def start_store(i):
            cp = pltpu.make_async_copy(
                bufs[i % _NBUF],
                kout_hbm.at[pl.ds(r0 + i * _CHUNK_ROWS, _CHUNK_ROWS)],
                sem_st)
            cp.start()
            stores[i] = cp

        for i in range(min(_NBUF, nchunk)):
            start_load(i)
        for i in range(nchunk):
            loads[i].wait()
            start_store(i)
            nxt = i + _NBUF
            if nxt < nchunk:
                stores[i].wait()
                start_load(nxt)
        for i in range(max(0, nchunk - _NBUF), nchunk):
            stores[i].wait()

        pos_vec = pos_v[...]
        scs = []
        for j in range(slabs_per):
            idx = pos_vec + (base + j) * S
            sk = pltpu.make_async_copy(
                kv_v.at[pl.ds(j * Q, Q)], kout_hbm.at[idx], sem_sc)
            sk.start()
            scs.append(sk)
        for c in scs:
            c.wait()

    return body(pos, k_val2, k_cache2)


def _tc_v_path(pos, v_val2, v_cache2, *, n_slabs, S, Q, D):
    block_rows = _TC_SLABS * S

    def body(pos_ref, vv, vc, vo):
        vo[...] = vc[...]
        for j in range(_TC_SLABS):
            for q in range(Q):
                vo[pl.ds(j * S + pos_ref[q], 1), :] = vv[pl.ds(j * Q + q, 1), :]

    grid_spec = pltpu.PrefetchScalarGridSpec(
        num_scalar_prefetch=1,
        grid=(n_slabs // _TC_SLABS,),
        in_specs=[
            pl.BlockSpec((_TC_SLABS * Q, D), lambda i, pos_ref: (i, 0)),
            pl.BlockSpec((block_rows, D), lambda i, pos_ref: (i, 0)),
        ],
        out_specs=pl.BlockSpec((block_rows, D), lambda i, pos_ref: (i, 0)),
    )
    return pl.pallas_call(
        body,
        grid_spec=grid_spec,
        out_shape=jax.ShapeDtypeStruct((n_slabs * S, D), jnp.float32),
        compiler_params=pltpu.CompilerParams(
            dimension_semantics=("arbitrary",)),
    )(pos, v_val2, v_cache2)


def kernel(input_pos, k_val, v_val, k_cache, v_cache):
    B, H, Q, D = k_val.shape
    S = k_cache.shape[2]
    n_slabs = B * H
    rows = n_slabs * S
    pos = input_pos.astype(jnp.int32)

    k_out = _sc_k_path(
        pos, k_val.reshape(n_slabs * Q, D), k_cache.reshape(rows, D),
        n_slabs=n_slabs, S=S, Q=Q, D=D)
    v_out = _tc_v_path(
        pos, v_val.reshape(n_slabs * Q, D), v_cache.reshape(rows, D),
        n_slabs=n_slabs, S=S, Q=Q, D=D)
    return (k_out.reshape(B, H, S, D), v_out.reshape(B, H, S, D))


# speedup vs baseline: 1.1250x; 1.0380x over previous
import functools

import jax
import jax.numpy as jnp
from jax import lax
from jax.experimental import pallas as pl
from jax.experimental.pallas import tpu as pltpu
from jax.experimental.pallas import tpu_sc as plsc

_NUM_CORES = 2
_NUM_SUBCORES = 16
_NUM_WORKERS = _NUM_CORES * _NUM_SUBCORES
_BLOCK_ROWS = 8192


def _tc_bulk_copy(k_cache2, v_cache2, *, rows, D):
    def body(kc, vc, ko, vo):
        ko[...] = kc[...]
        vo[...] = vc[...]

    spec = pl.BlockSpec((_BLOCK_ROWS, D), lambda i: (i, 0))
    return pl.pallas_call(
        body,
        grid=(rows // _BLOCK_ROWS,),
        in_specs=[spec, spec],
        out_specs=[spec, spec],
        out_shape=[jax.ShapeDtypeStruct((rows, D), jnp.float32)] * 2,
        compiler_params=pltpu.CompilerParams(
            dimension_semantics=("parallel",)),
    )(k_cache2, v_cache2)


def _sc_scatter(pos, k_val2, v_val2, k_ref, v_ref, *, n_slabs, S, Q, D):
    slabs_per = n_slabs // _NUM_WORKERS
    nval = slabs_per * Q
    mesh = plsc.VectorSubcoreMesh(
        core_axis_name="c", subcore_axis_name="s",
        num_cores=_NUM_CORES, num_subcores=_NUM_SUBCORES)

    @functools.partial(
        pl.kernel,
        out_type=(),
        mesh=mesh,
        scratch_types=[
            pltpu.VMEM((Q,), jnp.int32),
            pltpu.VMEM((nval, D), jnp.float32),
            pltpu.VMEM((nval, D), jnp.float32),
            pltpu.SemaphoreType.DMA,
            pltpu.SemaphoreType.DMA,
        ],
    )
    def body(pos_hbm, kval_hbm, vval_hbm, kout_hbm, vout_hbm,
             pos_v, kv_v, vv_v, sem_val, sem_sc):
        wid = lax.axis_index("s") * _NUM_CORES + lax.axis_index("c")
        base = wid * slabs_per

        lk = pltpu.make_async_copy(
            kval_hbm.at[pl.ds(base * Q, nval)], kv_v, sem_val)
        lv = pltpu.make_async_copy(
            vval_hbm.at[pl.ds(base * Q, nval)], vv_v, sem_val)
        lk.start()
        lv.start()
        pltpu.sync_copy(pos_hbm, pos_v)
        lk.wait()
        lv.wait()

        pos_vec = pos_v[...]
        scs = []
        for j in range(slabs_per):
            idx = pos_vec + (base + j) * S
            sk = pltpu.make_async_copy(
                kv_v.at[pl.ds(j * Q, Q)], kout_hbm.at[idx], sem_sc)
            sv = pltpu.make_async_copy(
                vv_v.at[pl.ds(j * Q, Q)], vout_hbm.at[idx], sem_sc)
            sk.start()
            sv.start()
            scs.append(sk)
            scs.append(sv)
        for c in scs:
            c.wait()

    body(pos, k_val2, v_val2, k_ref, v_ref)


def kernel(input_pos, k_val, v_val, k_cache, v_cache):
    B, H, Q, D = k_val.shape
    S = k_cache.shape[2]
    n_slabs = B * H
    rows = n_slabs * S
    pos = input_pos.astype(jnp.int32)

    k_copy, v_copy = _tc_bulk_copy(
        k_cache.reshape(rows, D), v_cache.reshape(rows, D), rows=rows, D=D)
    k_ref = jax.new_ref(k_copy)
    v_ref = jax.new_ref(v_copy)
    _sc_scatter(
        pos, k_val.reshape(n_slabs * Q, D), v_val.reshape(n_slabs * Q, D),
        k_ref, v_ref, n_slabs=n_slabs, S=S, Q=Q, D=D)
    return (k_ref[...].reshape(B, H, S, D), v_ref[...].reshape(B, H, S, D))
